# Initial kernel scaffold; baseline (speedup 1.0000x reference)
import jax
import jax.numpy as jnp
from jax.experimental import pallas as pl


def kernel(x, edge_index, W, att_src, att_dst):
    raise NotImplementedError("write your pallas kernel here")


# speedup vs baseline: 7.4019x; 7.4019x over previous
import functools

import jax
import jax.numpy as jnp
from jax import lax
from jax.experimental import pallas as pl
from jax.experimental.pallas import tpu as pltpu
from jax.experimental.pallas import tpu_sc as plsc

N = 10000
NP = 10240
NR = NP // 128
E = 320000
D = 128
NEG = 0.2
NTILES = 16
NSC = 2
RSLICE = NR // NTILES
EC = 80
EC3 = 64
EREM = 16
HTILE = N // NTILES
HCH = 25
ROWB = 5
RB = NP // ROWB


def _pre_body(x_ref, w_ref, asrc_ref, adst_ref, wx_ref, ssrc_ref, sdst_ref):
    xb = x_ref[...]
    w = w_ref[...]
    wx = lax.dot_general(xb, w, (((1,), (1,)), ((), ())),
                         preferred_element_type=jnp.float32)
    wx_ref[...] = wx
    ssrc_ref[...] = jnp.sum(wx * asrc_ref[...], axis=1)
    sdst_ref[...] = jnp.sum(wx * adst_ref[...], axis=1)


_tc_pre = pl.pallas_call(
    _pre_body,
    grid=(ROWB,),
    in_specs=[
        pl.BlockSpec((RB, D), lambda i: (i, 0)),
        pl.BlockSpec((D, D), lambda i: (0, 0)),
        pl.BlockSpec((1, D), lambda i: (0, 0)),
        pl.BlockSpec((1, D), lambda i: (0, 0)),
    ],
    out_specs=[
        pl.BlockSpec((RB, D), lambda i: (i, 0)),
        pl.BlockSpec((RB,), lambda i: (i,)),
        pl.BlockSpec((RB,), lambda i: (i,)),
    ],
    out_shape=[
        jax.ShapeDtypeStruct((NP, D), jnp.float32),
        jax.ShapeDtypeStruct((NP,), jnp.float32),
        jax.ShapeDtypeStruct((NP,), jnp.float32),
    ],
)


def _post_body(h2_ref, o_ref):
    hsum = h2_ref[0] + h2_ref[1]
    o_ref[...] = jnp.where(hsum > 0, hsum, jnp.exp(hsum) - 1.0)


_tc_post = pl.pallas_call(
    _post_body,
    grid=(ROWB,),
    in_specs=[pl.BlockSpec((2, 2000, D), lambda i: (0, i, 0))],
    out_specs=pl.BlockSpec((2000, D), lambda i: (i, 0)),
    out_shape=jax.ShapeDtypeStruct((N, D), jnp.float32),
)


def _gather2d(ref, idx):
    return plsc.load_gather(ref, [jnp.right_shift(idx, 7),
                                  jnp.bitwise_and(idx, 127)])


def _edge_scores(ssrc_v, sdst_v, si, di):
    e = _gather2d(sdst_v, di) + _gather2d(ssrc_v, si)
    return jnp.where(e >= 0, e, NEG * e)


def _sc_body(esrc_hbm, edst_hbm, ssrc_hbm, sdst_hbm, wx_hbm,
             out_hbm, stage_hbm, comb_hbm,
             ssrc_v, sdst_v, marr, sarr, red_v,
             esrc_v, edst_v, src3_v, dst3_v, srcr_v, dstr_v, alpha_v,
             rows_v, h_sh, sem):
    c = lax.axis_index("c")
    s = lax.axis_index("s")

    pltpu.sync_copy(ssrc_hbm, ssrc_v)
    pltpu.sync_copy(sdst_hbm, sdst_v)

    def _init(i, _):
        marr[i // 8, pl.ds((i % 8) * 16, 16)] = jnp.full((16,), -3.4e38,
                                                         jnp.float32)
        sarr[i // 8, pl.ds((i % 8) * 16, 16)] = jnp.zeros((16,), jnp.float32)
        return 0
    lax.fori_loop(0, NR * 8, _init, 0)

    def _zrows(i, _):
        rows_v[i // 8, pl.ds((i % 8) * 16, 16)] = jnp.zeros((16,), jnp.float32)
        return 0
    lax.fori_loop(0, EC3 * 8, _zrows, 0)

    def _zh(i, _):
        pltpu.sync_copy(rows_v.at[pl.ds(0, HCH)],
                        h_sh.at[pl.ds(s * HTILE + i * HCH, HCH)])
        return 0
    lax.fori_loop(0, HTILE // HCH, _zh, 0)

    ebase = s * (E // NTILES)

    def _p1(ci, _):
        base = ebase + ci * EC
        pltpu.sync_copy(esrc_hbm.at[pl.ds(base, EC)], esrc_v)
        pltpu.sync_copy(edst_hbm.at[pl.ds(base, EC)], edst_v)

        def _grp(g, _):
            si = esrc_v[pl.ds(g * 16, 16)]
            di = edst_v[pl.ds(g * 16, 16)]
            f = _edge_scores(ssrc_v, sdst_v, si, di)
            dhi = jnp.right_shift(di, 7)
            dlo = jnp.bitwise_and(di, 127)
            cur = plsc.load_gather(marr, [dhi, dlo])

            def _cond(cur):
                return jnp.any(f > cur)

            def _body(cur):
                plsc.store_scatter(marr, [dhi, dlo], jnp.maximum(f, cur),
                                   mask=f > cur)
                return plsc.load_gather(marr, [dhi, dlo])

            lax.while_loop(_cond, _body, cur)
            return 0
        lax.fori_loop(0, EC // 16, _grp, 0)
        return 0
    lax.fori_loop(0, (E // NTILES) // EC, _p1, 0)

    pltpu.sync_copy(marr, stage_hbm.at[c, s])
    plsc.subcore_barrier()
    rb = s * RSLICE
    pltpu.sync_copy(stage_hbm.at[c, 0, pl.ds(rb, RSLICE)], red_v)

    def _redmax(a, _):
        pltpu.sync_copy(stage_hbm.at[c, a, pl.ds(rb, RSLICE)],
                        rows_v.at[pl.ds(0, RSLICE)])
        for r in range(RSLICE):
            for k in range(8):
                red_v[r, pl.ds(k * 16, 16)] = jnp.maximum(
                    red_v[r, pl.ds(k * 16, 16)],
                    rows_v[r, pl.ds(k * 16, 16)])
        return 0
    lax.fori_loop(1, NTILES, _redmax, 0)
    pltpu.sync_copy(red_v, comb_hbm.at[c, pl.ds(rb, RSLICE)])
    plsc.subcore_barrier()
    pltpu.sync_copy(comb_hbm.at[c], marr)

    def _p2(ci, _):
        base = ebase + ci * EC
        pltpu.sync_copy(esrc_hbm.at[pl.ds(base, EC)], esrc_v)
        pltpu.sync_copy(edst_hbm.at[pl.ds(base, EC)], edst_v)

        def _grp(g, _):
            si = esrc_v[pl.ds(g * 16, 16)]
            di = edst_v[pl.ds(g * 16, 16)]
            f = _edge_scores(ssrc_v, sdst_v, si, di)
            dhi = jnp.right_shift(di, 7)
            dlo = jnp.bitwise_and(di, 127)
            m = plsc.load_gather(marr, [dhi, dlo])
            plsc.addupdate_scatter(sarr, [dhi, dlo], jnp.exp(f - m))
            return 0
        lax.fori_loop(0, EC // 16, _grp, 0)
        return 0
    lax.fori_loop(0, (E // NTILES) // EC, _p2, 0)

    pltpu.sync_copy(sarr, stage_hbm.at[c, s])
    plsc.subcore_barrier()
    pltpu.sync_copy(stage_hbm.at[c, 0, pl.ds(rb, RSLICE)], red_v)

    def _redsum(a, _):
        pltpu.sync_copy(stage_hbm.at[c, a, pl.ds(rb, RSLICE)],
                        rows_v.at[pl.ds(0, RSLICE)])
        for r in range(RSLICE):
            for k in range(8):
                red_v[r, pl.ds(k * 16, 16)] = (
                    red_v[r, pl.ds(k * 16, 16)]
                    + rows_v[r, pl.ds(k * 16, 16)])
        return 0
    lax.fori_loop(1, NTILES, _redsum, 0)
    pltpu.sync_copy(red_v, comb_hbm.at[c, pl.ds(rb, RSLICE)])
    plsc.subcore_barrier()
    pltpu.sync_copy(comb_hbm.at[c], sarr)

    ebase3 = (c * NTILES + s) * (E // (NSC * NTILES))

    def _alpha16(si, di):
        f = _edge_scores(ssrc_v, sdst_v, si, di)
        dhi = jnp.right_shift(di, 7)
        dlo = jnp.bitwise_and(di, 127)
        m = plsc.load_gather(marr, [dhi, dlo])
        ssum = plsc.load_gather(sarr, [dhi, dlo])
        return jnp.exp(f - m) / (ssum + 1e-16)

    def _p3(ci, _):
        base = ebase3 + ci * EC3
        pltpu.sync_copy(esrc_hbm.at[pl.ds(base, EC3)], src3_v)
        pltpu.sync_copy(edst_hbm.at[pl.ds(base, EC3)], dst3_v)

        def _ga(g, _):
            si = src3_v[pl.ds(g * 16, 16)]
            di = dst3_v[pl.ds(g * 16, 16)]
            alpha_v[pl.ds(g * 16, 16)] = _alpha16(si, di)
            return 0
        lax.fori_loop(0, EC3 // 16, _ga, 0)

        pltpu.async_copy(wx_hbm.at[src3_v], rows_v, sem).wait()

        def _scale(g, _):
            av = alpha_v[pl.ds(g * 16, 16)]
            for r2 in range(16):
                row = g * 16 + r2
                ab = jnp.full((16,), av[r2], jnp.float32)
                for cc in range(D // 16):
                    rows_v[row, pl.ds(cc * 16, 16)] = (
                        rows_v[row, pl.ds(cc * 16, 16)] * ab)
            return 0
        lax.fori_loop(0, EC3 // 16, _scale, 0)

        pltpu.sync_copy(rows_v, h_sh.at[dst3_v], add=True)
        return 0
    lax.fori_loop(0, (E // (NSC * NTILES)) // EC3, _p3, 0)

    baser = ebase3 + (E // (NSC * NTILES)) // EC3 * EC3
    pltpu.sync_copy(esrc_hbm.at[pl.ds(baser, EREM)], srcr_v)
    pltpu.sync_copy(edst_hbm.at[pl.ds(baser, EREM)], dstr_v)
    ar = _alpha16(srcr_v[...], dstr_v[...])
    pltpu.async_copy(wx_hbm.at[srcr_v], rows_v.at[pl.ds(0, EREM)], sem).wait()
    for r2 in range(EREM):
        ab = jnp.full((16,), ar[r2], jnp.float32)
        for cc in range(D // 16):
            rows_v[r2, pl.ds(cc * 16, 16)] = rows_v[r2, pl.ds(cc * 16, 16)] * ab
    pltpu.sync_copy(rows_v.at[pl.ds(0, EREM)], h_sh.at[dstr_v], add=True)

    plsc.subcore_barrier()

    def _out(i, _):
        pltpu.sync_copy(h_sh.at[pl.ds(s * HTILE + i * HCH, HCH)],
                        rows_v.at[pl.ds(0, HCH)])
        pltpu.sync_copy(rows_v.at[pl.ds(0, HCH)],
                        out_hbm.at[c, pl.ds(s * HTILE + i * HCH, HCH)])
        return 0
    lax.fori_loop(0, HTILE // HCH, _out, 0)


_sc_main = functools.partial(
    pl.kernel,
    mesh=plsc.VectorSubcoreMesh(core_axis_name="c", subcore_axis_name="s"),
    out_type=[
        jax.ShapeDtypeStruct((NSC, N, D), jnp.float32),
        jax.ShapeDtypeStruct((NSC, NTILES, NR, D), jnp.float32),
        jax.ShapeDtypeStruct((NSC, NR, D), jnp.float32),
    ],
    compiler_params=pltpu.CompilerParams(needs_layout_passes=False,
                                         use_tc_tiling_on_sc=False),
    scratch_types=[
        pltpu.VMEM((NR, D), jnp.float32),
        pltpu.VMEM((NR, D), jnp.float32),
        pltpu.VMEM((NR, D), jnp.float32),
        pltpu.VMEM((NR, D), jnp.float32),
        pltpu.VMEM((RSLICE, D), jnp.float32),
        pltpu.VMEM((EC,), jnp.int32),
        pltpu.VMEM((EC,), jnp.int32),
        pltpu.VMEM((EC3,), jnp.int32),
        pltpu.VMEM((EC3,), jnp.int32),
        pltpu.VMEM((EREM,), jnp.int32),
        pltpu.VMEM((EREM,), jnp.int32),
        pltpu.VMEM((EC3,), jnp.float32),
        pltpu.VMEM((EC3, D), jnp.float32),
        pltpu.VMEM_SHARED((N, D), jnp.float32),
        pltpu.SemaphoreType.DMA,
    ],
)(_sc_body)


def kernel(x, edge_index, W, att_src, att_dst):
    xp = jnp.pad(x, ((0, NP - N), (0, 0)))
    wx, ssrc, sdst = _tc_pre(xp, W, att_src, att_dst)
    h2, _, _ = _sc_main(edge_index[0], edge_index[1],
                        ssrc.reshape(NR, D), sdst.reshape(NR, D), wx)
    out = _tc_post(h2)
    return out
